# baseline (speedup 1.0000x reference)
import jax
import jax.numpy as jnp
from jax import lax
from jax.experimental import pallas as pl
from jax.experimental.pallas import tpu as pltpu
from jax.experimental.pallas import tpu_sc as plsc

EMB_K = 16
BATCH = 16384
_ROWS_PER_BLOCK = 128 // EMB_K

_NC = 2
_NS = 16
_NW = _NC * _NS
_BPW = BATCH // _NW
_CHUNK = 128
_NCHUNK = _BPW // _CHUNK


def _body(x_ref, w_ref, h_ref, out_ref, xv, uidx, vidx, ublk, vblk,
          wu0, wu1, wv0, wv1, outv, sem0, sem1):
    wid = lax.axis_index("s") * _NC + lax.axis_index("c")
    base = wid * _BPW

    pltpu.sync_copy(x_ref.at[pl.ds(base * 2, _BPW * 2)], xv)

    iota = lax.iota(jnp.int32, 16)

    for j in range(_BPW // 16):
        pairs = j * 32 + iota * 2
        u16 = plsc.load_gather(xv, [pairs])
        v16 = plsc.load_gather(xv, [pairs + 1])
        sl = pl.ds(j * 16, 16)
        uidx[sl] = u16
        vidx[sl] = v16
        ublk[sl] = u16 >> 3
        vblk[sl] = v16 >> 3

    bufs = [(wu0, wv0, sem0), (wu1, wv1, sem1)]

    def fire(c):
        wu, wv, sem = bufs[c % 2]
        sl = pl.ds(c * _CHUNK, _CHUNK)
        return (pltpu.async_copy(w_ref.at[ublk.at[sl]], wu, sem),
                pltpu.async_copy(h_ref.at[vblk.at[sl]], wv, sem))

    one = jnp.full((16,), 1.0, jnp.float32)
    inflight = fire(0)

    for c in range(_NCHUNK):
        nxt = fire(c + 1) if c + 1 < _NCHUNK else None
        for cp in inflight:
            cp.wait()
        inflight = nxt
        wu, wv, _ = bufs[c % 2]

        for b in range(_CHUNK // 16):
            i0 = c * _CHUNK + b * 16
            sl = pl.ds(i0, 16)
            rows = b * 16 + iota
            offu = (uidx[sl] & 7) * EMB_K
            offv = (vidx[sl] & 7) * EMB_K
            acc = jnp.zeros((16,), jnp.float32)
            for k in range(EMB_K):
                u = plsc.load_gather(wu, [rows, offu + k])
                v = plsc.load_gather(wv, [rows, offv + k])
                acc = acc + u * v
            outv[sl] = one / (one + jnp.exp(-acc))

    pltpu.sync_copy(outv, out_ref.at[pl.ds(base, _BPW)])


@jax.jit
def _mf_sc(x, W, H):
    mesh = plsc.VectorSubcoreMesh(core_axis_name="c", subcore_axis_name="s")
    nblk = W.shape[0] // _ROWS_PER_BLOCK
    return pl.kernel(
        _body,
        mesh=mesh,
        compiler_params=pltpu.CompilerParams(needs_layout_passes=False),
        out_type=jax.ShapeDtypeStruct((BATCH,), jnp.float32),
        scratch_types=[
            pltpu.VMEM((_BPW * 2,), jnp.int32),
            pltpu.VMEM((_BPW,), jnp.int32),
            pltpu.VMEM((_BPW,), jnp.int32),
            pltpu.VMEM((_BPW,), jnp.int32),
            pltpu.VMEM((_BPW,), jnp.int32),
            pltpu.VMEM((_CHUNK, 128), jnp.float32),
            pltpu.VMEM((_CHUNK, 128), jnp.float32),
            pltpu.VMEM((_CHUNK, 128), jnp.float32),
            pltpu.VMEM((_CHUNK, 128), jnp.float32),
            pltpu.VMEM((_BPW,), jnp.float32),
            pltpu.SemaphoreType.DMA,
            pltpu.SemaphoreType.DMA,
        ],
    )(x.reshape(-1), W.reshape(nblk, 128), H.reshape(nblk, 128))


def kernel(x, W, H):
    return _mf_sc(x, W, H)
